# baseline (speedup 1.0000x reference)
import functools

import jax
import jax.numpy as jnp
from jax import lax
from jax.experimental import pallas as pl
from jax.experimental.pallas import tpu as pltpu
from jax.experimental.pallas import tpu_sc as plsc

_EMBED_DIM = 64
_SEQ = 550
_N_IDX = 16384

_NC = 2
_NS = 16
_NW = _NC * _NS
_PER_W = _N_IDX // _NW
_K = 64
_NCHUNK = _PER_W // _K
_NLANE = 16
_NSEG = _EMBED_DIM // _NLANE


def _sc_partial_sums(x1d, table):
  mesh = plsc.VectorSubcoreMesh(core_axis_name="c", subcore_axis_name="s")

  @functools.partial(
      pl.kernel,
      out_type=jax.ShapeDtypeStruct((_NW * 2 * _EMBED_DIM,), jnp.float32),
      mesh=mesh,
      scratch_types=[
          pltpu.VMEM((_PER_W,), jnp.int32),
          pltpu.VMEM((2, _K, _EMBED_DIM), jnp.float32),
          pltpu.VMEM((2 * _EMBED_DIM,), jnp.float32),
          pltpu.SemaphoreType.DMA,
          pltpu.SemaphoreType.DMA,
          pltpu.SemaphoreType.DMA,
      ],
      compiler_params=pltpu.CompilerParams(use_tc_tiling_on_sc=True),
  )
  def k(x_hbm, table_hbm, out_hbm, idx_v, rows_v, acc_v, sem0, sem1, sem_i):
    cid = lax.axis_index("c")
    sid = lax.axis_index("s")
    wid = sid * _NC + cid
    base = wid * _PER_W
    pltpu.async_copy(x_hbm.at[pl.ds(base, _PER_W)], idx_v, sem_i).wait()

    sems = (sem0, sem1)

    def enqueue(c, buf):
      @pl.loop(0, _K // _NLANE)
      def _(g):
        v = idx_v[pl.ds(c * _K + g * _NLANE, _NLANE)]
        for l in range(_NLANE):
          pltpu.async_copy(
              table_hbm.at[v[l]], rows_v.at[buf, g * _NLANE + l], sems[buf]
          )

    def drain(buf):
      pltpu.make_async_copy(
          table_hbm.at[pl.ds(0, _K)], rows_v.at[buf], sems[buf]
      ).wait()

    def row_adder(buf):
      def row_add(j, accs):
        return tuple(
            accs[s] + rows_v[buf, j, pl.ds(s * _NLANE, _NLANE)]
            for s in range(_NSEG)
        )
      return row_add

    zero = jnp.zeros((_NLANE,), jnp.float32)
    acc0 = (zero,) * _NSEG
    acc1 = (zero,) * _NSEG

    enqueue(0, 0)
    for c in range(_NCHUNK):
      buf = c & 1
      if c + 1 < _NCHUNK:
        enqueue(c + 1, (c + 1) & 1)
      drain(buf)
      g = base + c * _K
      n0 = jnp.clip(_SEQ - g, 0, _K)
      acc0 = lax.fori_loop(0, n0, row_adder(buf), acc0)
      acc1 = lax.fori_loop(n0, _K, row_adder(buf), acc1)

    for s in range(_NSEG):
      acc_v[pl.ds(s * _NLANE, _NLANE)] = acc0[s]
      acc_v[pl.ds(_EMBED_DIM + s * _NLANE, _NLANE)] = acc1[s]
    pltpu.sync_copy(acc_v, out_hbm.at[pl.ds(wid * 2 * _EMBED_DIM,
                                            2 * _EMBED_DIM)])

  return k(x1d, table)


def _tc_finish(partials, fc1_w, fc1_b):
  def body(p_ref, w_ref, b_ref, o_ref):
    p = p_ref[...].reshape(_NW, 2 * _EMBED_DIM)
    s = jnp.sum(p, axis=0, keepdims=True)
    w = w_ref[...]
    d0 = jnp.sum(s[:, :_EMBED_DIM] * w) * (1.0 / _SEQ)
    d1 = jnp.sum(s[:, _EMBED_DIM:] * w) * (1.0 / (_N_IDX - _SEQ))
    b = b_ref[0, 0]
    sig0 = 1.0 / (1.0 + jnp.exp(-(d0 + b)))
    sig1 = 1.0 / (1.0 + jnp.exp(-(d1 + b)))
    o_ref[...] = jnp.broadcast_to(0.5 * (sig0 + sig1), (1, 1))

  return pl.pallas_call(
      body,
      out_shape=jax.ShapeDtypeStruct((1, 1), jnp.float32),
  )(partials, fc1_w, fc1_b.reshape(1, 1))


def kernel(x, table, fc1_w, fc1_b):
  x1d = x.astype(jnp.int32).reshape(_N_IDX)
  partials = _sc_partial_sums(x1d, table)
  out = _tc_finish(partials, fc1_w, fc1_b)
  return out[0, 0]


# speedup vs baseline: 1.0736x; 1.0736x over previous
import dataclasses
import functools

import jax
import jax.numpy as jnp
from jax import lax
from jax.experimental import pallas as pl
from jax.experimental.pallas import tpu as pltpu
from jax.experimental.pallas import tpu_sc as plsc

_EMBED_DIM = 64
_SEQ = 550
_N_IDX = 16384

_NC = 2
_NS = 16
_NLANE = 16

_BLK_N = 2048
_P_LEN = 49 * _BLK_N
_SLICE = _P_LEN // _NS
_B_POS = _P_LEN - _NLANE
_NVREG = _N_IDX // _NLANE
_SPLIT_VREG = _SEQ // _NLANE
_SPLIT_LANE = _SEQ % _NLANE


def _tc_project(table_t, fc1_w, fc1_b):
  n_steps = _P_LEN // _BLK_N

  def body(t_ref, w_ref, b_ref, p_ref):
    p_ref[...] = jnp.dot(
        w_ref[...], t_ref[...], preferred_element_type=jnp.float32
    )

    @pl.when(pl.program_id(0) == n_steps - 1)
    def _():
      p_ref[0, pl.ds(_BLK_N - _NLANE, _NLANE)] = jnp.broadcast_to(
          b_ref[0, 0], (_NLANE,)
      )

  return pl.pallas_call(
      body,
      grid=(n_steps,),
      in_specs=[
          pl.BlockSpec((_EMBED_DIM, _BLK_N), lambda j: (0, j)),
          pl.BlockSpec((1, _EMBED_DIM), lambda j: (0, 0)),
          pl.BlockSpec((1, 1), lambda j: (0, 0), memory_space=pltpu.SMEM),
      ],
      out_specs=pl.BlockSpec((1, _BLK_N), lambda j: (0, j)),
      out_shape=jax.ShapeDtypeStruct((1, _P_LEN), jnp.float32),
  )(table_t, fc1_w, fc1_b.reshape(1, 1))


def _sc_bag_sigmoid(x1d, p1d):
  mesh = plsc.VectorSubcoreMesh(core_axis_name="c", subcore_axis_name="s")
  cp = pltpu.CompilerParams()
  if "needs_layout_passes" in pltpu.CompilerParams.__dataclass_fields__:
    cp = dataclasses.replace(cp, needs_layout_passes=False)

  @functools.partial(
      pl.kernel,
      out_type=jax.ShapeDtypeStruct((_NC * _NLANE,), jnp.float32),
      mesh=mesh,
      compiler_params=cp,
      scratch_types=[
          pltpu.VMEM((_SLICE,), jnp.float32),
          pltpu.VMEM((_N_IDX,), jnp.int32),
          pltpu.VMEM((2, _NLANE), jnp.float32),
          pltpu.VMEM((_NS, 2, _NLANE), jnp.float32),
          pltpu.VMEM((_NLANE,), jnp.float32),
          pltpu.VMEM_SHARED((_NS, 2, _NLANE), jnp.float32),
          pltpu.SemaphoreType.DMA,
      ],
  )
  def k(x_hbm, p_hbm, out_hbm, p_v, x_v, acc_v, all_v, res_v, shared, sem):
    cid = lax.axis_index("c")
    sid = lax.axis_index("s")
    lo = sid * _SLICE
    pltpu.async_copy(p_hbm.at[pl.ds(lo, _SLICE)], p_v, sem).wait()
    pltpu.async_copy(x_hbm, x_v, sem).wait()

    lane = lax.iota(jnp.int32, 16)
    zero = jnp.zeros((_NLANE,), jnp.float32)

    def contrib(j, lane_mask=None):
      idxv = x_v[pl.ds(j * _NLANE, _NLANE)]
      m = (idxv >= lo) & (idxv < lo + _SLICE)
      if lane_mask is not None:
        m = m & lane_mask
      local = idxv - lo
      g = plsc.load_gather(p_v, [local], mask=m)
      return jnp.where(m, g, 0.0)

    acc0 = lax.fori_loop(
        0, _SPLIT_VREG, lambda j, a: a + contrib(j), zero
    )
    acc0 = acc0 + contrib(_SPLIT_VREG, lane < _SPLIT_LANE)
    acc1 = contrib(_SPLIT_VREG, lane >= _SPLIT_LANE)
    acc1 = lax.fori_loop(
        _SPLIT_VREG + 1, _NVREG, lambda j, a: a + contrib(j), acc1
    )

    acc_v[0, pl.ds(0, _NLANE)] = acc0
    acc_v[1, pl.ds(0, _NLANE)] = acc1
    pltpu.sync_copy(acc_v, shared.at[sid])
    plsc.subcore_barrier()

    @pl.when(sid == _NS - 1)
    def _():
      pltpu.sync_copy(shared, all_v)

      def add_row(s, carry):
        a0, a1 = carry
        return (
            a0 + all_v[s, 0, pl.ds(0, _NLANE)],
            a1 + all_v[s, 1, pl.ds(0, _NLANE)],
        )

      t0, t1 = lax.fori_loop(0, _NS, add_row, (zero, zero))
      s0 = jnp.sum(t0)
      s1 = jnp.sum(t1)
      bvec = p_v[pl.ds(_B_POS - (_NS - 1) * _SLICE, _NLANE)]
      b = bvec[0]
      z0 = s0 * (1.0 / _SEQ) + b
      z1 = s1 * (1.0 / (_N_IDX - _SEQ)) + b
      sig0 = 1.0 / (1.0 + jnp.exp(jnp.broadcast_to(-z0, (_NLANE,))))
      sig1 = 1.0 / (1.0 + jnp.exp(jnp.broadcast_to(-z1, (_NLANE,))))
      res_v[...] = 0.5 * (sig0 + sig1)
      pltpu.sync_copy(res_v, out_hbm.at[pl.ds(cid * _NLANE, _NLANE)])

  return k(x1d, p1d)


def kernel(x, table, fc1_w, fc1_b):
  x1d = x.astype(jnp.int32).reshape(_N_IDX)
  p = _tc_project(table.T, fc1_w, fc1_b)
  out = _sc_bag_sigmoid(x1d, p.reshape(_P_LEN))
  return out[0]


# speedup vs baseline: 1.5765x; 1.4684x over previous
import dataclasses
import functools

import jax
import jax.numpy as jnp
from jax import lax
from jax.experimental import pallas as pl
from jax.experimental.pallas import tpu as pltpu
from jax.experimental.pallas import tpu_sc as plsc

_EMBED_DIM = 64
_SEQ = 550
_N_IDX = 16384

_NC = 2
_NS = 16
_NLANE = 16

_BLK_N = 8192
_P_LEN = 13 * _BLK_N
_SLICE = _P_LEN // _NS
_B_POS = _P_LEN - _NLANE
_NVREG = _N_IDX // _NLANE
_SPLIT_VREG = _SEQ // _NLANE
_SPLIT_LANE = _SEQ % _NLANE


def _tc_project(table_t, fc1_w, fc1_b):
  n_steps = _P_LEN // _BLK_N

  def body(t_ref, w_ref, b_ref, p_ref):
    p_ref[...] = jnp.dot(
        w_ref[...], t_ref[...], preferred_element_type=jnp.float32
    )

    @pl.when(pl.program_id(0) == n_steps - 1)
    def _():
      p_ref[0, pl.ds(_BLK_N - _NLANE, _NLANE)] = jnp.broadcast_to(
          b_ref[0, 0], (_NLANE,)
      )

  return pl.pallas_call(
      body,
      grid=(n_steps,),
      in_specs=[
          pl.BlockSpec((_EMBED_DIM, _BLK_N), lambda j: (0, j)),
          pl.BlockSpec((1, _EMBED_DIM), lambda j: (0, 0)),
          pl.BlockSpec((1, 1), lambda j: (0, 0), memory_space=pltpu.SMEM),
      ],
      out_specs=pl.BlockSpec((1, _BLK_N), lambda j: (0, j)),
      out_shape=jax.ShapeDtypeStruct((1, _P_LEN), jnp.float32),
  )(table_t, fc1_w, fc1_b.reshape(1, 1))


def _sc_bag_sigmoid(x1d, p1d):
  mesh = plsc.VectorSubcoreMesh(core_axis_name="c", subcore_axis_name="s")
  cp = pltpu.CompilerParams()
  if "needs_layout_passes" in pltpu.CompilerParams.__dataclass_fields__:
    cp = dataclasses.replace(cp, needs_layout_passes=False)

  @functools.partial(
      pl.kernel,
      out_type=jax.ShapeDtypeStruct((_NC * _NLANE,), jnp.float32),
      mesh=mesh,
      compiler_params=cp,
      scratch_types=[
          pltpu.VMEM((_SLICE,), jnp.float32),
          pltpu.VMEM((_N_IDX,), jnp.int32),
          pltpu.VMEM((2, _NLANE), jnp.float32),
          pltpu.VMEM((_NS, 2, _NLANE), jnp.float32),
          pltpu.VMEM((_NLANE,), jnp.float32),
          pltpu.VMEM_SHARED((_NS, 2, _NLANE), jnp.float32),
          pltpu.SemaphoreType.DMA,
      ],
  )
  def k(x_hbm, p_hbm, out_hbm, p_v, x_v, acc_v, all_v, res_v, shared, sem):
    cid = lax.axis_index("c")
    sid = lax.axis_index("s")
    lo = sid * _SLICE
    pltpu.async_copy(p_hbm.at[pl.ds(lo, _SLICE)], p_v, sem).wait()
    pltpu.async_copy(x_hbm, x_v, sem).wait()

    lane = lax.iota(jnp.int32, 16)
    zero = jnp.zeros((_NLANE,), jnp.float32)

    def contrib(j, lane_mask=None):
      idxv = x_v[pl.ds(j * _NLANE, _NLANE)]
      m = (idxv >= lo) & (idxv < lo + _SLICE)
      if lane_mask is not None:
        m = m & lane_mask
      local = idxv - lo
      g = plsc.load_gather(p_v, [local], mask=m)
      return jnp.where(m, g, 0.0)

    acc0 = lax.fori_loop(
        0, _SPLIT_VREG, lambda j, a: a + contrib(j), zero
    )
    acc0 = acc0 + contrib(_SPLIT_VREG, lane < _SPLIT_LANE)
    acc1 = contrib(_SPLIT_VREG, lane >= _SPLIT_LANE)
    acc1 = lax.fori_loop(
        _SPLIT_VREG + 1, _NVREG, lambda j, a: a + contrib(j), acc1
    )

    acc_v[0, pl.ds(0, _NLANE)] = acc0
    acc_v[1, pl.ds(0, _NLANE)] = acc1
    pltpu.sync_copy(acc_v, shared.at[sid])
    plsc.subcore_barrier()

    @pl.when(sid == _NS - 1)
    def _():
      pltpu.sync_copy(shared, all_v)

      def add_row(s, carry):
        a0, a1 = carry
        return (
            a0 + all_v[s, 0, pl.ds(0, _NLANE)],
            a1 + all_v[s, 1, pl.ds(0, _NLANE)],
        )

      t0, t1 = lax.fori_loop(0, _NS, add_row, (zero, zero))
      s0 = jnp.sum(t0)
      s1 = jnp.sum(t1)
      bvec = p_v[pl.ds(_B_POS - (_NS - 1) * _SLICE, _NLANE)]
      b = bvec[0]
      z0 = s0 * (1.0 / _SEQ) + b
      z1 = s1 * (1.0 / (_N_IDX - _SEQ)) + b
      sig0 = 1.0 / (1.0 + jnp.exp(jnp.broadcast_to(-z0, (_NLANE,))))
      sig1 = 1.0 / (1.0 + jnp.exp(jnp.broadcast_to(-z1, (_NLANE,))))
      res_v[...] = 0.5 * (sig0 + sig1)
      pltpu.sync_copy(res_v, out_hbm.at[pl.ds(cid * _NLANE, _NLANE)])

  return k(x1d, p1d)


def kernel(x, table, fc1_w, fc1_b):
  x1d = x.astype(jnp.int32).reshape(_N_IDX)
  p = _tc_project(table.T, fc1_w, fc1_b)
  out = _sc_bag_sigmoid(x1d, p.reshape(_P_LEN))
  return out[0]


# speedup vs baseline: 1.8112x; 1.1488x over previous
import dataclasses
import functools

import jax
import jax.numpy as jnp
from jax import lax
from jax.experimental import pallas as pl
from jax.experimental.pallas import tpu as pltpu
from jax.experimental.pallas import tpu_sc as plsc

_EMBED_DIM = 64
_SEQ = 550
_N_IDX = 16384

_NC = 2
_NS = 16
_NLANE = 16

_BLK_N = 16384
_P_LEN = 7 * _BLK_N
_SLICE = _P_LEN // _NS
_B_POS = _P_LEN - _NLANE
_NVREG = _N_IDX // _NLANE
_SPLIT_VREG = _SEQ // _NLANE
_SPLIT_LANE = _SEQ % _NLANE


def _tc_project(table_t, fc1_w, fc1_b):
  n_steps = _P_LEN // _BLK_N

  def body(t_ref, w_ref, b_ref, p_ref):
    p_ref[...] = jnp.dot(
        w_ref[...], t_ref[...], preferred_element_type=jnp.float32
    )

    @pl.when(pl.program_id(0) == n_steps - 1)
    def _():
      p_ref[0, pl.ds(_BLK_N - _NLANE, _NLANE)] = jnp.broadcast_to(
          b_ref[0, 0], (_NLANE,)
      )

  return pl.pallas_call(
      body,
      grid=(n_steps,),
      in_specs=[
          pl.BlockSpec((_EMBED_DIM, _BLK_N), lambda j: (0, j)),
          pl.BlockSpec((1, _EMBED_DIM), lambda j: (0, 0)),
          pl.BlockSpec((1, 1), lambda j: (0, 0), memory_space=pltpu.SMEM),
      ],
      out_specs=pl.BlockSpec((1, _BLK_N), lambda j: (0, j)),
      out_shape=jax.ShapeDtypeStruct((1, _P_LEN), jnp.float32),
  )(table_t, fc1_w, fc1_b.reshape(1, 1))


def _sc_bag_sigmoid(x1d, p1d):
  mesh = plsc.VectorSubcoreMesh(core_axis_name="c", subcore_axis_name="s")
  cp = pltpu.CompilerParams()
  if "needs_layout_passes" in pltpu.CompilerParams.__dataclass_fields__:
    cp = dataclasses.replace(cp, needs_layout_passes=False)

  @functools.partial(
      pl.kernel,
      out_type=jax.ShapeDtypeStruct((_NC * _NLANE,), jnp.float32),
      mesh=mesh,
      compiler_params=cp,
      scratch_types=[
          pltpu.VMEM((_SLICE,), jnp.float32),
          pltpu.VMEM((_N_IDX,), jnp.int32),
          pltpu.VMEM((2, _NLANE), jnp.float32),
          pltpu.VMEM((_NS, 2, _NLANE), jnp.float32),
          pltpu.VMEM((_NLANE,), jnp.float32),
          pltpu.VMEM_SHARED((_NS, 2, _NLANE), jnp.float32),
          pltpu.SemaphoreType.DMA,
      ],
  )
  def k(x_hbm, p_hbm, out_hbm, p_v, x_v, acc_v, all_v, res_v, shared, sem):
    cid = lax.axis_index("c")
    sid = lax.axis_index("s")
    lo = sid * _SLICE
    pltpu.async_copy(p_hbm.at[pl.ds(lo, _SLICE)], p_v, sem).wait()
    pltpu.async_copy(x_hbm, x_v, sem).wait()

    lane = lax.iota(jnp.int32, 16)
    zero = jnp.zeros((_NLANE,), jnp.float32)

    def contrib(j, lane_mask=None):
      idxv = x_v[pl.ds(j * _NLANE, _NLANE)]
      local = idxv - lo
      m = local.astype(jnp.uint32) < jnp.uint32(_SLICE)
      if lane_mask is not None:
        m = m & lane_mask
      g = plsc.load_gather(p_v, [local], mask=m)
      return jnp.where(m, g, 0.0)

    def unrolled4(start, a):
      for u in range(4):
        a = a + contrib(start + u)
      return a

    acc0 = lax.fori_loop(0, 8, lambda j, a: unrolled4(j * 4, a), zero)
    acc0 = acc0 + contrib(32) + contrib(33)
    acc0 = acc0 + contrib(_SPLIT_VREG, lane < _SPLIT_LANE)
    acc1 = contrib(_SPLIT_VREG, lane >= _SPLIT_LANE) + contrib(35)
    acc1 = lax.fori_loop(
        0, (_NVREG - 36) // 4, lambda j, a: unrolled4(36 + j * 4, a), acc1
    )

    acc_v[0, pl.ds(0, _NLANE)] = acc0
    acc_v[1, pl.ds(0, _NLANE)] = acc1
    pltpu.sync_copy(acc_v, shared.at[sid])
    plsc.subcore_barrier()

    @pl.when(sid == _NS - 1)
    def _():
      pltpu.sync_copy(shared, all_v)

      def add_row(s, carry):
        a0, a1 = carry
        return (
            a0 + all_v[s, 0, pl.ds(0, _NLANE)],
            a1 + all_v[s, 1, pl.ds(0, _NLANE)],
        )

      t0, t1 = lax.fori_loop(0, _NS, add_row, (zero, zero))
      s0 = jnp.sum(t0)
      s1 = jnp.sum(t1)
      bvec = p_v[pl.ds(_B_POS - (_NS - 1) * _SLICE, _NLANE)]
      b = bvec[0]
      z0 = s0 * (1.0 / _SEQ) + b
      z1 = s1 * (1.0 / (_N_IDX - _SEQ)) + b
      sig0 = 1.0 / (1.0 + jnp.exp(jnp.broadcast_to(-z0, (_NLANE,))))
      sig1 = 1.0 / (1.0 + jnp.exp(jnp.broadcast_to(-z1, (_NLANE,))))
      res_v[...] = 0.5 * (sig0 + sig1)
      pltpu.sync_copy(res_v, out_hbm.at[pl.ds(cid * _NLANE, _NLANE)])

  return k(x1d, p1d)


def kernel(x, table, fc1_w, fc1_b):
  x1d = x.astype(jnp.int32).reshape(_N_IDX)
  p = _tc_project(table.T, fc1_w, fc1_b)
  out = _sc_bag_sigmoid(x1d, p.reshape(_P_LEN))
  return out[0]


# speedup vs baseline: 1.8583x; 1.0260x over previous
import dataclasses
import functools

import jax
import jax.numpy as jnp
from jax import lax
from jax.experimental import pallas as pl
from jax.experimental.pallas import tpu as pltpu
from jax.experimental.pallas import tpu_sc as plsc

_EMBED_DIM = 64
_SEQ = 550
_N_IDX = 16384

_NC = 2
_NS = 16
_NLANE = 16

_BLK_N = 28672
_P_LEN = 4 * _BLK_N
_SLICE = _P_LEN // _NS
_B_POS = _P_LEN - _NLANE
_NVREG = _N_IDX // _NLANE
_SPLIT_VREG = _SEQ // _NLANE
_SPLIT_LANE = _SEQ % _NLANE


def _tc_project(table_t, fc1_w, fc1_b):
  n_steps = _P_LEN // _BLK_N

  def body(t_ref, w_ref, b_ref, p_ref):
    p_ref[...] = jnp.dot(
        w_ref[...], t_ref[...], preferred_element_type=jnp.float32
    )

    @pl.when(pl.program_id(0) == n_steps - 1)
    def _():
      p_ref[0, pl.ds(_BLK_N - _NLANE, _NLANE)] = jnp.broadcast_to(
          b_ref[0, 0], (_NLANE,)
      )

  return pl.pallas_call(
      body,
      grid=(n_steps,),
      in_specs=[
          pl.BlockSpec((_EMBED_DIM, _BLK_N), lambda j: (0, j)),
          pl.BlockSpec((1, _EMBED_DIM), lambda j: (0, 0)),
          pl.BlockSpec((1, 1), lambda j: (0, 0), memory_space=pltpu.SMEM),
      ],
      out_specs=pl.BlockSpec((1, _BLK_N), lambda j: (0, j)),
      out_shape=jax.ShapeDtypeStruct((1, _P_LEN), jnp.float32),
  )(table_t, fc1_w, fc1_b.reshape(1, 1))


def _sc_bag_sigmoid(x1d, p1d):
  mesh = plsc.VectorSubcoreMesh(core_axis_name="c", subcore_axis_name="s")
  cp = pltpu.CompilerParams()
  if "needs_layout_passes" in pltpu.CompilerParams.__dataclass_fields__:
    cp = dataclasses.replace(cp, needs_layout_passes=False)

  @functools.partial(
      pl.kernel,
      out_type=jax.ShapeDtypeStruct((_NC * _NLANE,), jnp.float32),
      mesh=mesh,
      compiler_params=cp,
      scratch_types=[
          pltpu.VMEM((_SLICE,), jnp.float32),
          pltpu.VMEM((_N_IDX,), jnp.int32),
          pltpu.VMEM((2, _NLANE), jnp.float32),
          pltpu.VMEM((_NS, 2, _NLANE), jnp.float32),
          pltpu.VMEM((_NLANE,), jnp.float32),
          pltpu.VMEM_SHARED((_NS, 2, _NLANE), jnp.float32),
          pltpu.SemaphoreType.DMA,
      ],
  )
  def k(x_hbm, p_hbm, out_hbm, p_v, x_v, acc_v, all_v, res_v, shared, sem):
    cid = lax.axis_index("c")
    sid = lax.axis_index("s")
    lo = sid * _SLICE
    pltpu.async_copy(p_hbm.at[pl.ds(lo, _SLICE)], p_v, sem).wait()
    pltpu.async_copy(x_hbm, x_v, sem).wait()

    lane = lax.iota(jnp.int32, 16)
    zero = jnp.zeros((_NLANE,), jnp.float32)

    def contrib(j, lane_mask=None):
      idxv = x_v[pl.ds(j * _NLANE, _NLANE)]
      local = idxv - lo
      m = local.astype(jnp.uint32) < jnp.uint32(_SLICE)
      if lane_mask is not None:
        m = m & lane_mask
      g = plsc.load_gather(p_v, [local], mask=m)
      return jnp.where(m, g, 0.0)

    def unrolled4(start, accs):
      return tuple(a + contrib(start + u) for u, a in enumerate(accs))

    zeros4 = (zero,) * 4
    a4 = lax.fori_loop(0, 8, lambda j, a: unrolled4(j * 4, a), zeros4)
    acc0 = (a4[0] + a4[1]) + (a4[2] + a4[3])
    acc0 = acc0 + contrib(32) + contrib(33)
    acc0 = acc0 + contrib(_SPLIT_VREG, lane < _SPLIT_LANE)
    b4 = lax.fori_loop(
        0, (_NVREG - 36) // 4, lambda j, a: unrolled4(36 + j * 4, a), zeros4
    )
    acc1 = (b4[0] + b4[1]) + (b4[2] + b4[3])
    acc1 = acc1 + contrib(_SPLIT_VREG, lane >= _SPLIT_LANE) + contrib(35)

    acc_v[0, pl.ds(0, _NLANE)] = acc0
    acc_v[1, pl.ds(0, _NLANE)] = acc1
    pltpu.sync_copy(acc_v, shared.at[sid])
    plsc.subcore_barrier()

    @pl.when(sid == _NS - 1)
    def _():
      pltpu.sync_copy(shared, all_v)

      def add_row(s, carry):
        a0, a1 = carry
        return (
            a0 + all_v[s, 0, pl.ds(0, _NLANE)],
            a1 + all_v[s, 1, pl.ds(0, _NLANE)],
        )

      t0, t1 = lax.fori_loop(0, _NS, add_row, (zero, zero))
      s0 = jnp.sum(t0)
      s1 = jnp.sum(t1)
      bvec = p_v[pl.ds(_B_POS - (_NS - 1) * _SLICE, _NLANE)]
      b = bvec[0]
      z0 = s0 * (1.0 / _SEQ) + b
      z1 = s1 * (1.0 / (_N_IDX - _SEQ)) + b
      sig0 = 1.0 / (1.0 + jnp.exp(jnp.broadcast_to(-z0, (_NLANE,))))
      sig1 = 1.0 / (1.0 + jnp.exp(jnp.broadcast_to(-z1, (_NLANE,))))
      res_v[...] = 0.5 * (sig0 + sig1)
      pltpu.sync_copy(res_v, out_hbm.at[pl.ds(cid * _NLANE, _NLANE)])

  return k(x1d, p1d)


def kernel(x, table, fc1_w, fc1_b):
  x1d = x.astype(jnp.int32).reshape(_N_IDX)
  p = _tc_project(table.T, fc1_w, fc1_b)
  out = _sc_bag_sigmoid(x1d, p.reshape(_P_LEN))
  return out[0]


# speedup vs baseline: 1.9727x; 1.0616x over previous
import dataclasses
import functools

import jax
import jax.numpy as jnp
from jax import lax
from jax.experimental import pallas as pl
from jax.experimental.pallas import tpu as pltpu
from jax.experimental.pallas import tpu_sc as plsc

_EMBED_DIM = 64
_SEQ = 550
_N_IDX = 16384

_NC = 2
_NS = 16
_NLANE = 16

_BLK_N = 28672
_P_LEN = 4 * _BLK_N
_NPART = 8
_SLICE = _P_LEN // _NPART
_B_POS = _P_LEN - _NLANE
_NVREG = _N_IDX // _NLANE
_HVREG = _NVREG // 2
_SPLIT_VREG = _SEQ // _NLANE
_SPLIT_LANE = _SEQ % _NLANE


def _tc_project(table_t, fc1_w, fc1_b):
  n_steps = _P_LEN // _BLK_N

  def body(t_ref, w_ref, b_ref, p_ref):
    p_ref[...] = jnp.dot(
        w_ref[...], t_ref[...], preferred_element_type=jnp.float32
    )

    @pl.when(pl.program_id(0) == n_steps - 1)
    def _():
      p_ref[0, pl.ds(_BLK_N - _NLANE, _NLANE)] = jnp.broadcast_to(
          b_ref[0, 0], (_NLANE,)
      )

  return pl.pallas_call(
      body,
      grid=(n_steps,),
      in_specs=[
          pl.BlockSpec((_EMBED_DIM, _BLK_N), lambda j: (0, j)),
          pl.BlockSpec((1, _EMBED_DIM), lambda j: (0, 0)),
          pl.BlockSpec((1, 1), lambda j: (0, 0), memory_space=pltpu.SMEM),
      ],
      out_specs=pl.BlockSpec((1, _BLK_N), lambda j: (0, j)),
      out_shape=jax.ShapeDtypeStruct((1, _P_LEN), jnp.float32),
  )(table_t, fc1_w, fc1_b.reshape(1, 1))


def _sc_bag_sigmoid(x1d, p1d):
  mesh = plsc.VectorSubcoreMesh(core_axis_name="c", subcore_axis_name="s")
  cp = pltpu.CompilerParams()
  if "needs_layout_passes" in pltpu.CompilerParams.__dataclass_fields__:
    cp = dataclasses.replace(cp, needs_layout_passes=False)

  @functools.partial(
      pl.kernel,
      out_type=jax.ShapeDtypeStruct((_NC * _NLANE,), jnp.float32),
      mesh=mesh,
      compiler_params=cp,
      scratch_types=[
          pltpu.VMEM((_SLICE,), jnp.float32),
          pltpu.VMEM((_N_IDX // 2,), jnp.int32),
          pltpu.VMEM((2, _NLANE), jnp.float32),
          pltpu.VMEM((_NS, 2, _NLANE), jnp.float32),
          pltpu.VMEM((_NLANE,), jnp.float32),
          pltpu.VMEM_SHARED((_NS, 2, _NLANE), jnp.float32),
          pltpu.SemaphoreType.DMA,
      ],
  )
  def k(x_hbm, p_hbm, out_hbm, p_v, x_v, acc_v, all_v, res_v, shared, sem):
    cid = lax.axis_index("c")
    sid = lax.axis_index("s")
    part = sid // 2
    half = sid % 2
    lo = part * _SLICE
    pltpu.async_copy(p_hbm.at[pl.ds(lo, _SLICE)], p_v, sem).wait()
    pltpu.async_copy(
        x_hbm.at[pl.ds(half * (_N_IDX // 2), _N_IDX // 2)], x_v, sem
    ).wait()

    lane = lax.iota(jnp.int32, 16)
    zero = jnp.zeros((_NLANE,), jnp.float32)
    zeros4 = (zero,) * 4

    def contrib(j, lane_mask=None):
      idxv = x_v[pl.ds(j * _NLANE, _NLANE)]
      local = idxv - lo
      m = local.astype(jnp.uint32) < jnp.uint32(_SLICE)
      if lane_mask is not None:
        m = m & lane_mask
      g = plsc.load_gather(p_v, [local], mask=m)
      return jnp.where(m, g, 0.0)

    def unrolled4(start, accs):
      return tuple(a + contrib(start + u) for u, a in enumerate(accs))

    def sum4(a4):
      return (a4[0] + a4[1]) + (a4[2] + a4[3])

    @pl.when(half == 0)
    def _():
      a4 = lax.fori_loop(0, 8, lambda j, a: unrolled4(j * 4, a), zeros4)
      acc0 = sum4(a4) + contrib(32) + contrib(33)
      acc0 = acc0 + contrib(_SPLIT_VREG, lane < _SPLIT_LANE)
      b4 = lax.fori_loop(
          0, (_HVREG - 36) // 4, lambda j, a: unrolled4(36 + j * 4, a), zeros4
      )
      acc1 = sum4(b4)
      acc1 = acc1 + contrib(_SPLIT_VREG, lane >= _SPLIT_LANE) + contrib(35)
      acc_v[0, pl.ds(0, _NLANE)] = acc0
      acc_v[1, pl.ds(0, _NLANE)] = acc1

    @pl.when(half == 1)
    def _():
      b4 = lax.fori_loop(
          0, _HVREG // 4, lambda j, a: unrolled4(j * 4, a), zeros4
      )
      acc_v[0, pl.ds(0, _NLANE)] = zero
      acc_v[1, pl.ds(0, _NLANE)] = sum4(b4)
    pltpu.sync_copy(acc_v, shared.at[sid])
    plsc.subcore_barrier()

    @pl.when(sid == _NS - 1)
    def _():
      pltpu.sync_copy(shared, all_v)

      def add_row(s, carry):
        a0, a1 = carry
        return (
            a0 + all_v[s, 0, pl.ds(0, _NLANE)],
            a1 + all_v[s, 1, pl.ds(0, _NLANE)],
        )

      t0, t1 = lax.fori_loop(0, _NS, add_row, (zero, zero))
      s0 = jnp.sum(t0)
      s1 = jnp.sum(t1)
      bvec = p_v[pl.ds(_B_POS - (_NPART - 1) * _SLICE, _NLANE)]
      b = bvec[0]
      z0 = s0 * (1.0 / _SEQ) + b
      z1 = s1 * (1.0 / (_N_IDX - _SEQ)) + b
      sig0 = 1.0 / (1.0 + jnp.exp(jnp.broadcast_to(-z0, (_NLANE,))))
      sig1 = 1.0 / (1.0 + jnp.exp(jnp.broadcast_to(-z1, (_NLANE,))))
      res_v[...] = 0.5 * (sig0 + sig1)
      pltpu.sync_copy(res_v, out_hbm.at[pl.ds(cid * _NLANE, _NLANE)])

  return k(x1d, p1d)


def kernel(x, table, fc1_w, fc1_b):
  x1d = x.astype(jnp.int32).reshape(_N_IDX)
  p = _tc_project(table.T, fc1_w, fc1_b)
  out = _sc_bag_sigmoid(x1d, p.reshape(_P_LEN))
  return out[0]


# speedup vs baseline: 2.0099x; 1.0189x over previous
import dataclasses
import functools

import jax
import jax.numpy as jnp
from jax import lax
from jax.experimental import pallas as pl
from jax.experimental.pallas import tpu as pltpu
from jax.experimental.pallas import tpu_sc as plsc

_EMBED_DIM = 64
_SEQ = 550
_N_IDX = 16384

_NC = 2
_NS = 16
_NLANE = 16

_BLK_N = 28672
_P_LEN = 4 * _BLK_N
_NPART = 8
_SLICE = _P_LEN // _NPART
_B_POS = _P_LEN - _NLANE
_NVREG = _N_IDX // _NLANE
_HVREG = _NVREG // 2
_SPLIT_VREG = _SEQ // _NLANE
_SPLIT_LANE = _SEQ % _NLANE


def _tc_project(table_t, fc1_w, fc1_b):
  n_steps = _P_LEN // _BLK_N

  def body(t_ref, w_ref, b_ref, p_ref):
    p_ref[...] = jnp.dot(
        w_ref[...], t_ref[...], preferred_element_type=jnp.float32
    )

    @pl.when(pl.program_id(0) == n_steps - 1)
    def _():
      p_ref[0, pl.ds(_BLK_N - _NLANE, _NLANE)] = jnp.broadcast_to(
          b_ref[0, 0], (_NLANE,)
      )

  return pl.pallas_call(
      body,
      grid=(n_steps,),
      in_specs=[
          pl.BlockSpec((_EMBED_DIM, _BLK_N), lambda j: (0, j)),
          pl.BlockSpec((1, _EMBED_DIM), lambda j: (0, 0)),
          pl.BlockSpec((1, 1), lambda j: (0, 0), memory_space=pltpu.SMEM),
      ],
      out_specs=pl.BlockSpec((1, _BLK_N), lambda j: (0, j)),
      out_shape=jax.ShapeDtypeStruct((1, _P_LEN), jnp.float32),
  )(table_t, fc1_w, fc1_b.reshape(1, 1))


def _sc_bag_sigmoid(x1d, p1d):
  mesh = plsc.VectorSubcoreMesh(core_axis_name="c", subcore_axis_name="s")
  cp = pltpu.CompilerParams()
  if "needs_layout_passes" in pltpu.CompilerParams.__dataclass_fields__:
    cp = dataclasses.replace(cp, needs_layout_passes=False)

  @functools.partial(
      pl.kernel,
      out_type=jax.ShapeDtypeStruct((_NC * _NLANE,), jnp.float32),
      mesh=mesh,
      compiler_params=cp,
      scratch_types=[
          pltpu.VMEM((_SLICE,), jnp.float32),
          pltpu.VMEM((_N_IDX // 2,), jnp.int32),
          pltpu.VMEM((2, _NLANE), jnp.float32),
          pltpu.VMEM((_NS, 2, _NLANE), jnp.float32),
          pltpu.VMEM((_NLANE,), jnp.float32),
          pltpu.VMEM_SHARED((_NS, 2, _NLANE), jnp.float32),
          pltpu.SemaphoreType.DMA,
          pltpu.SemaphoreType.DMA,
      ],
  )
  def k(x_hbm, p_hbm, out_hbm, p_v, x_v, acc_v, all_v, res_v, shared, sem,
        sem2):
    cid = lax.axis_index("c")
    sid = lax.axis_index("s")
    part = sid // 2
    half = sid % 2
    lo = part * _SLICE
    cp_p = pltpu.async_copy(p_hbm.at[pl.ds(lo, _SLICE)], p_v, sem)
    cp_x = pltpu.async_copy(
        x_hbm.at[pl.ds(half * (_N_IDX // 2), _N_IDX // 2)], x_v, sem2
    )
    cp_p.wait()
    cp_x.wait()

    lane = lax.iota(jnp.int32, 16)
    zero = jnp.zeros((_NLANE,), jnp.float32)
    zeros4 = (zero,) * 4

    def contrib(j, lane_mask=None):
      idxv = x_v[pl.ds(j * _NLANE, _NLANE)]
      local = idxv - lo
      m = local.astype(jnp.uint32) < jnp.uint32(_SLICE)
      if lane_mask is not None:
        m = m & lane_mask
      g = plsc.load_gather(p_v, [local], mask=m)
      return jnp.where(m, g, 0.0)

    def unrolled4(start, accs):
      return tuple(a + contrib(start + u) for u, a in enumerate(accs))

    def sum4(a4):
      return (a4[0] + a4[1]) + (a4[2] + a4[3])

    @pl.when(half == 0)
    def _():
      a4 = lax.fori_loop(0, 8, lambda j, a: unrolled4(j * 4, a), zeros4)
      acc0 = sum4(a4) + contrib(32) + contrib(33)
      acc0 = acc0 + contrib(_SPLIT_VREG, lane < _SPLIT_LANE)
      b4 = lax.fori_loop(
          0, (_HVREG - 36) // 4, lambda j, a: unrolled4(36 + j * 4, a), zeros4
      )
      acc1 = sum4(b4)
      acc1 = acc1 + contrib(_SPLIT_VREG, lane >= _SPLIT_LANE) + contrib(35)
      acc_v[0, pl.ds(0, _NLANE)] = acc0
      acc_v[1, pl.ds(0, _NLANE)] = acc1

    @pl.when(half == 1)
    def _():
      b4 = lax.fori_loop(
          0, _HVREG // 4, lambda j, a: unrolled4(j * 4, a), zeros4
      )
      acc_v[0, pl.ds(0, _NLANE)] = zero
      acc_v[1, pl.ds(0, _NLANE)] = sum4(b4)
    pltpu.sync_copy(acc_v, shared.at[sid])
    plsc.subcore_barrier()

    @pl.when(sid == _NS - 1)
    def _():
      pltpu.sync_copy(shared, all_v)

      def add_row(s, carry):
        a0, a1 = carry
        return (
            a0 + all_v[s, 0, pl.ds(0, _NLANE)],
            a1 + all_v[s, 1, pl.ds(0, _NLANE)],
        )

      t0, t1 = lax.fori_loop(0, _NS, add_row, (zero, zero))
      s0 = jnp.sum(t0)
      s1 = jnp.sum(t1)
      bvec = p_v[pl.ds(_B_POS - (_NPART - 1) * _SLICE, _NLANE)]
      b = bvec[0]
      z0 = s0 * (1.0 / _SEQ) + b
      z1 = s1 * (1.0 / (_N_IDX - _SEQ)) + b
      sig0 = 1.0 / (1.0 + jnp.exp(jnp.broadcast_to(-z0, (_NLANE,))))
      sig1 = 1.0 / (1.0 + jnp.exp(jnp.broadcast_to(-z1, (_NLANE,))))
      res_v[...] = 0.5 * (sig0 + sig1)
      pltpu.sync_copy(res_v, out_hbm.at[pl.ds(cid * _NLANE, _NLANE)])

  return k(x1d, p1d)


def kernel(x, table, fc1_w, fc1_b):
  x1d = x.astype(jnp.int32).reshape(_N_IDX)
  p = _tc_project(table.T, fc1_w, fc1_b)
  out = _sc_bag_sigmoid(x1d, p.reshape(_P_LEN))
  return out[0]
